# Initial kernel scaffold; baseline (speedup 1.0000x reference)
import jax
import jax.numpy as jnp
from jax.experimental import pallas as pl


def kernel(x, edge_index, W3, b3, W4, b4):
    raise NotImplementedError("write your pallas kernel here")


# speedup vs baseline: 14.1151x; 14.1151x over previous
import jax
import jax.numpy as jnp
from jax import lax
from jax.experimental import pallas as pl
from jax.experimental.pallas import tpu as pltpu
from jax.experimental.pallas import tpu_sc as plsc

N_NODES = 10000
N_EDGES = 320000
NP = 10240
NC, NS = 2, 16
NT = NC * NS
CH = 128
CH_TILE = (N_EDGES + NT * CH - 1) // (NT * CH)
E_PAD = NT * CH_TILE * CH
ROWS_SUB = NP // NS

_f32 = jnp.float32
_mesh = plsc.VectorSubcoreMesh(
    core_axis_name="c", subcore_axis_name="s", num_cores=NC, num_subcores=NS)


def _deg_body(cols_hbm, degp_hbm, cols_v, ones_v, zb_v, acc):
    c = lax.axis_index("c")
    s = lax.axis_index("s")
    w = c * NS + s
    pltpu.sync_copy(cols_hbm.at[w], cols_v)
    for i in range(CH):
        ones_v[i, :] = jnp.ones((16,), _f32)
        zb_v[i, :] = jnp.zeros((16,), _f32)
    for k in range(ROWS_SUB // CH):
        pltpu.sync_copy(zb_v, acc.at[pl.ds(s * ROWS_SUB + k * CH, CH)])
    plsc.subcore_barrier()

    def body(j, carry):
        pltpu.sync_copy(ones_v, acc.at[cols_v.at[j]], add=True)
        return carry

    lax.fori_loop(0, CH_TILE, body, 0)
    plsc.subcore_barrier()
    for k in range(ROWS_SUB // CH):
        sl = pl.ds(s * ROWS_SUB + k * CH, CH)
        pltpu.sync_copy(acc.at[sl], degp_hbm.at[c, sl])


_deg_call = pl.kernel(
    _deg_body,
    out_type=jax.ShapeDtypeStruct((NC, NP, 16), _f32),
    mesh=_mesh,
    scratch_types=[
        pltpu.VMEM((CH_TILE, CH), jnp.int32),
        pltpu.VMEM((CH, 16), _f32),
        pltpu.VMEM((CH, 16), _f32),
        pltpu.VMEM_SHARED((NP, 16), _f32),
    ],
)


def _edge_body(src_hbm, rows_hbm, cols_hbm, out_hbm,
               rows_v, cols_v, msg_v, zb_v, acc, gsem):
    c = lax.axis_index("c")
    s = lax.axis_index("s")
    w = c * NS + s
    pltpu.sync_copy(rows_hbm.at[w], rows_v)
    pltpu.sync_copy(cols_hbm.at[w], cols_v)
    for i in range(64):
        for j2 in range(8):
            zb_v[i, pl.ds(j2 * 16, 16)] = jnp.zeros((16,), _f32)
    for k in range(ROWS_SUB // 64):
        pltpu.sync_copy(zb_v, acc.at[pl.ds(s * ROWS_SUB + k * 64, 64)])
    plsc.subcore_barrier()

    def body(j, carry):
        pltpu.async_copy(src_hbm.at[rows_v.at[j]], msg_v, gsem).wait()
        pltpu.sync_copy(msg_v, acc.at[cols_v.at[j]], add=True)
        return carry

    lax.fori_loop(0, CH_TILE, body, 0)
    plsc.subcore_barrier()
    for k in range(ROWS_SUB // CH):
        sl = pl.ds(s * ROWS_SUB + k * CH, CH)
        pltpu.sync_copy(acc.at[sl], out_hbm.at[c, sl])


_edge_call = pl.kernel(
    _edge_body,
    out_type=jax.ShapeDtypeStruct((NC, NP, 128), _f32),
    mesh=_mesh,
    scratch_types=[
        pltpu.VMEM((CH_TILE, CH), jnp.int32),
        pltpu.VMEM((CH_TILE, CH), jnp.int32),
        pltpu.VMEM((CH, 128), _f32),
        pltpu.VMEM((64, 128), _f32),
        pltpu.VMEM_SHARED((NP, 128), _f32),
        pltpu.SemaphoreType.DMA,
    ],
)


BM1 = 2048


def _tc1_body(degp_ref, x_ref, xs_ref, disb_ref):
    d = degp_ref[0, :, 0:1] + degp_ref[1, :, 0:1] + 1.0
    dis = lax.rsqrt(d)
    disb = jnp.broadcast_to(dis, (BM1, 128))
    disb_ref[...] = disb
    xs_ref[...] = x_ref[...] * disb


_tc1 = pl.pallas_call(
    _tc1_body,
    grid=(NP // BM1,),
    in_specs=[
        pl.BlockSpec((NC, BM1, 16), lambda i: (0, i, 0)),
        pl.BlockSpec((BM1, 128), lambda i: (i, 0)),
    ],
    out_specs=[
        pl.BlockSpec((BM1, 128), lambda i: (i, 0)),
        pl.BlockSpec((BM1, 128), lambda i: (i, 0)),
    ],
    out_shape=[jax.ShapeDtypeStruct((NP, 128), _f32)] * 2,
)


BM3 = 1024


def _tc3_body(p_ref, xs_ref, disb_ref, w3_ref, b3_ref, w4_ref, ms_ref):
    agg = (p_ref[0] + p_ref[1] + xs_ref[...]) * disb_ref[...]
    h = lax.dot_general(agg, w3_ref[...], (((1,), (1,)), ((), ())),
                        preferred_element_type=_f32)
    h = jnp.maximum(h + b3_ref[...], 0.0)
    m = lax.dot_general(h, w4_ref[...], (((1,), (1,)), ((), ())),
                        preferred_element_type=_f32)
    ms_ref[...] = m * disb_ref[...]


_tc3 = pl.pallas_call(
    _tc3_body,
    grid=(NP // BM3,),
    in_specs=[
        pl.BlockSpec((NC, BM3, 128), lambda i: (0, i, 0)),
        pl.BlockSpec((BM3, 128), lambda i: (i, 0)),
        pl.BlockSpec((BM3, 128), lambda i: (i, 0)),
        pl.BlockSpec((256, 128), lambda i: (0, 0)),
        pl.BlockSpec((1, 256), lambda i: (0, 0)),
        pl.BlockSpec((128, 256), lambda i: (0, 0)),
    ],
    out_specs=pl.BlockSpec((BM3, 128), lambda i: (i, 0)),
    out_shape=jax.ShapeDtypeStruct((NP, 128), _f32),
)


def _tc5_body(q_ref, ms_ref, disb_ref, b4_ref, out_ref):
    out_ref[...] = ((q_ref[0] + q_ref[1] + ms_ref[...]) * disb_ref[...]
                    + b4_ref[...])


_tc5 = pl.pallas_call(
    _tc5_body,
    grid=(NP // BM3,),
    in_specs=[
        pl.BlockSpec((NC, BM3, 128), lambda i: (0, i, 0)),
        pl.BlockSpec((BM3, 128), lambda i: (i, 0)),
        pl.BlockSpec((BM3, 128), lambda i: (i, 0)),
        pl.BlockSpec((1, 128), lambda i: (0, 0)),
    ],
    out_specs=pl.BlockSpec((BM3, 128), lambda i: (i, 0)),
    out_shape=jax.ShapeDtypeStruct((NP, 128), _f32),
)


def kernel(x, edge_index, W3, b3, W4, b4):
    ei = edge_index.astype(jnp.int32)
    npad = E_PAD - ei.shape[1]
    pad = jnp.full((npad,), NP - 1, jnp.int32)
    rows3 = jnp.concatenate([ei[0], pad]).reshape(NT, CH_TILE, CH)
    cols3 = jnp.concatenate([ei[1], pad]).reshape(NT, CH_TILE, CH)
    x_pad = jnp.pad(x, ((0, NP - x.shape[0]), (0, 0)))

    degp = _deg_call(cols3)
    xs, disb = _tc1(degp, x_pad)
    p = _edge_call(xs, rows3, cols3)
    ms = _tc3(p, xs, disb, W3, b3.reshape(1, -1), W4)
    q = _edge_call(ms, rows3, cols3)
    out_pad = _tc5(q, ms, disb, b4.reshape(1, -1))
    return out_pad[:N_NODES]
